# baseline (speedup 1.0000x reference)
import functools

import jax
import jax.numpy as jnp
from jax import lax
from jax.experimental import pallas as pl
from jax.experimental.pallas import tpu as pltpu
from jax.experimental.pallas import tpu_sc as plsc

_EXTRA = 16
_CHUNK = 1600
_B = 96
_BLKN = 2000
_BLKE = 4000


def _h1_body(h_ref, w_ref, o_ref):
    o_ref[...] = lax.dot_general(h_ref[...], w_ref[...], (((1,), (1,)), ((), ())),
                                 preferred_element_type=jnp.float32)


def _e1_body(a_ref, w_ref, b_ref, o_ref):
    o_ref[...] = lax.dot_general(a_ref[...], w_ref[...], (((1,), (1,)), ((), ())),
                                 preferred_element_type=jnp.float32) + b_ref[...]


def _fin_body(hagg_ref, deg_ref, h_ref, w2_ref, b2_ref,
              wih_ref, bih_ref, whh_ref, bhh_ref, g_ref, be_ref, mu_ref, var_ref,
              o_ref):
    D = h_ref.shape[1]
    h = h_ref[...]
    agg = lax.dot_general(hagg_ref[...], w2_ref[...], (((1,), (1,)), ((), ())),
                          preferred_element_type=jnp.float32) + deg_ref[...] * b2_ref[...]
    gi = lax.dot_general(agg, wih_ref[...], (((1,), (1,)), ((), ())),
                         preferred_element_type=jnp.float32) + bih_ref[...]
    gh = lax.dot_general(h, whh_ref[...], (((1,), (1,)), ((), ())),
                         preferred_element_type=jnp.float32) + bhh_ref[...]
    r = jax.nn.sigmoid(gi[:, :D] + gh[:, :D])
    z = jax.nn.sigmoid(gi[:, D:2 * D] + gh[:, D:2 * D])
    n = jnp.tanh(gi[:, 2 * D:] + r * gh[:, 2 * D:])
    h_new = (1.0 - z) * n + z * h
    bn = g_ref[...] * (h_new - mu_ref[...]) * lax.rsqrt(var_ref[...] + 1e-5) + be_ref[...]
    o_ref[...] = h + bn


def _make_sc_agg(N, E, D):
    info = plsc.get_sparse_core_info()
    NC, NS, L = info.num_cores, info.num_subcores, info.num_lanes
    NW = NC * NS
    PASSES = 2
    NBUK = NW * PASSES
    RPT = ((N + NBUK - 1) // NBUK + 7) // 8 * 8
    NPAD = NBUK * RPT
    TRASHL = RPT
    B = 32
    KG = D // L
    mesh = plsc.VectorSubcoreMesh(core_axis_name="c", subcore_axis_name="s")

    @functools.partial(
        pl.kernel, mesh=mesh,
        out_type=jax.ShapeDtypeStruct((NPAD, D), jnp.float32),
        scratch_types=[
            pltpu.VMEM((2, B), jnp.int32),
            pltpu.VMEM((2, B), jnp.int32),
            pltpu.VMEM((2, B), jnp.int32),
            pltpu.VMEM((16,), jnp.int32),
            pltpu.VMEM((16,), jnp.int32),
            pltpu.VMEM((16,), jnp.int32),
            pltpu.VMEM((2, B, D), jnp.float32),
            pltpu.VMEM((2, B, D), jnp.float32),
            pltpu.VMEM((RPT + 1, D), jnp.float32),
            pltpu.SemaphoreType.DMA,
            pltpu.SemaphoreType.DMA,
            pltpu.SemaphoreType.DMA,
            pltpu.SemaphoreType.DMA,
        ],
    )
    def sc_agg(h1_hbm, e1_hbm, src_hbm, dst_hbm, eid_hbm, lo_hbm, hi_hbm, rb_hbm,
               out_hbm, gsrc, geid, sloc, lov, hiv, rbv, bufh, bufe, aggl,
               semh0, seme0, semh1, seme1):
        cid = lax.axis_index("c")
        sid = lax.axis_index("s")
        wid = sid * NC + cid
        iota = lax.iota(jnp.int32, L)
        zero16 = jnp.zeros((L,), jnp.float32)
        sems = ((semh0, seme0), (semh1, seme1))

        for paz in range(PASSES):
            buk = paz * NW + wid

            def zrow(rr, _):
                for k in range(KG):
                    aggl[rr, pl.ds(k * L, L)] = zero16
                return 0
            lax.fori_loop(0, RPT + 1, zrow, 0)

            pltpu.sync_copy(lo_hbm.at[pl.ds(buk * L, L)], lov)
            pltpu.sync_copy(hi_hbm.at[pl.ds(buk * L, L)], hiv)
            pltpu.sync_copy(rb_hbm.at[pl.ds(buk * L, L)], rbv)
            lovec = lov[pl.ds(0, L)]
            hivec = hiv[pl.ds(0, L)]
            rbvec = rbv[pl.ds(0, L)]
            lo16vec = lovec - lovec % L
            lo16 = pl.multiple_of(lo16vec[0], 16)
            hi = hivec[0]
            nb = (hi - lo16 + B - 1) // B

            def stage_fire(g, slot):
                base = pl.multiple_of(lo16 + g * B, 16)
                pltpu.sync_copy(src_hbm.at[pl.ds(base, B)], gsrc.at[slot])
                pltpu.sync_copy(eid_hbm.at[pl.ds(base, B)], geid.at[slot])
                pltpu.sync_copy(dst_hbm.at[pl.ds(base, B)], sloc.at[slot])
                for t in range(B // L):
                    giv = lo16vec + g * B + t * L + iota
                    valid = (giv >= lovec) & (giv < hivec)
                    dv = sloc[slot, pl.ds(t * L, L)]
                    sloc[slot, pl.ds(t * L, L)] = jnp.where(valid, dv - rbvec, TRASHL)
                pltpu.async_copy(h1_hbm.at[gsrc.at[slot]], bufh.at[slot], sems[slot][0])
                pltpu.async_copy(e1_hbm.at[geid.at[slot]], bufe.at[slot], sems[slot][1])

            def drain(slot):
                pltpu.make_async_copy(h1_hbm.at[gsrc.at[slot]], bufh.at[slot],
                                      sems[slot][0]).wait()
                pltpu.make_async_copy(e1_hbm.at[geid.at[slot]], bufe.at[slot],
                                      sems[slot][1]).wait()

            def accum(slot):
                def acc16(q, _):
                    locv = sloc[slot, pl.ds(q * L, L)]
                    for lane in range(L):
                        loc = locv[lane]
                        i = q * L + lane
                        for k in range(KG):
                            v = (bufh[slot, i, pl.ds(k * L, L)]
                                 + bufe[slot, i, pl.ds(k * L, L)])
                            a = aggl[loc, pl.ds(k * L, L)]
                            aggl[loc, pl.ds(k * L, L)] = a + jnp.maximum(v, 0.0)
                    return 0
                lax.fori_loop(0, B // L, acc16, 0)

            @pl.when(nb > 0)
            def _prime():
                stage_fire(0, 0)
                drain(0)

            def pair_body(g2, _):
                b0 = 2 * g2
                b1 = b0 + 1
                b2 = b0 + 2

                @pl.when(b1 < nb)
                def _s1():
                    stage_fire(b1, 1)
                accum(0)

                @pl.when(b1 < nb)
                def _w1():
                    drain(1)

                @pl.when(b2 < nb)
                def _s0():
                    stage_fire(b2, 0)

                @pl.when(b1 < nb)
                def _a1():
                    accum(1)

                @pl.when(b2 < nb)
                def _w0():
                    drain(0)
                return 0

            lax.fori_loop(0, (nb + 1) // 2, pair_body, 0)
            pltpu.sync_copy(aggl.at[pl.ds(0, RPT)], out_hbm.at[pl.ds(buk * RPT, RPT)])

    return sc_agg, NPAD, RPT


def kernel(h, edge_index, edge_attr, W1, b1, W2, b2, bn_gamma, bn_beta,
           bn_mean, bn_var, W_ih, W_hh, b_ih, b_hh):
    N, D = h.shape
    E, DE = edge_attr.shape
    ei = edge_index.astype(jnp.int32)
    src, dst = ei[0], ei[1]
    W1a = W1[:, :D]
    W1b = W1[:, D:]

    gN = N // _BLKN
    H1 = pl.pallas_call(
        _h1_body,
        grid=(gN,),
        in_specs=[pl.BlockSpec((_BLKN, D), lambda i: (i, 0)),
                  pl.BlockSpec((D, D), lambda i: (0, 0))],
        out_specs=pl.BlockSpec((_BLKN, D), lambda i: (i, 0)),
        out_shape=jax.ShapeDtypeStruct((N, D), jnp.float32),
    )(h, W1a)

    gE = E // _BLKE
    E1 = pl.pallas_call(
        _e1_body,
        grid=(gE,),
        in_specs=[pl.BlockSpec((_BLKE, DE), lambda i: (i, 0)),
                  pl.BlockSpec((D, DE), lambda i: (0, 0)),
                  pl.BlockSpec((1, D), lambda i: (0, 0))],
        out_specs=pl.BlockSpec((_BLKE, D), lambda i: (i, 0)),
        out_shape=jax.ShapeDtypeStruct((E, D), jnp.float32),
    )(edge_attr, W1b, b1.reshape(1, D))

    sc_agg, NPAD, RPT = _make_sc_agg(N, E, D)
    NBUK = NPAD // RPT
    CH = 1024
    perm = jnp.argsort(dst).astype(jnp.int32)
    dsts = dst[perm]
    pad = jnp.zeros((CH,), jnp.int32)
    srcp = jnp.concatenate([src[perm], pad])
    dstp = jnp.concatenate([dsts, pad])
    eidp = jnp.concatenate([perm, pad])
    noff = jnp.searchsorted(dsts, jnp.arange(NPAD + 1), side="left").astype(jnp.int32)
    off = noff[::RPT]
    lo_tab = jnp.repeat(off[:NBUK], 16)
    hi_tab = jnp.repeat(off[1:NBUK + 1], 16)
    rb_tab = jnp.repeat(jnp.arange(NBUK, dtype=jnp.int32) * RPT, 16)
    haggw = sc_agg(H1, E1, srcp, dstp, eidp, lo_tab, hi_tab, rb_tab)
    hagg0 = haggw[:N, :D]
    deg0 = (noff[1:N + 1] - noff[:N]).astype(jnp.float32).reshape(N, 1)

    D3 = 3 * D
    out = pl.pallas_call(
        _fin_body,
        grid=(gN,),
        in_specs=[pl.BlockSpec((_BLKN, D), lambda i: (i, 0)),
                  pl.BlockSpec((_BLKN, 1), lambda i: (i, 0)),
                  pl.BlockSpec((_BLKN, D), lambda i: (i, 0)),
                  pl.BlockSpec((D, D), lambda i: (0, 0)),
                  pl.BlockSpec((1, D), lambda i: (0, 0)),
                  pl.BlockSpec((D3, D), lambda i: (0, 0)),
                  pl.BlockSpec((1, D3), lambda i: (0, 0)),
                  pl.BlockSpec((D3, D), lambda i: (0, 0)),
                  pl.BlockSpec((1, D3), lambda i: (0, 0)),
                  pl.BlockSpec((1, D), lambda i: (0, 0)),
                  pl.BlockSpec((1, D), lambda i: (0, 0)),
                  pl.BlockSpec((1, D), lambda i: (0, 0)),
                  pl.BlockSpec((1, D), lambda i: (0, 0))],
        out_specs=pl.BlockSpec((_BLKN, D), lambda i: (i, 0)),
        out_shape=jax.ShapeDtypeStruct((N, D), jnp.float32),
    )(hagg0, deg0, h, W2, b2.reshape(1, D), W_ih, b_ih.reshape(1, D3),
      W_hh, b_hh.reshape(1, D3), bn_gamma.reshape(1, D), bn_beta.reshape(1, D),
      bn_mean.reshape(1, D), bn_var.reshape(1, D))
    return out


# speedup vs baseline: 2.9355x; 2.9355x over previous
import functools

import jax
import jax.numpy as jnp
from jax import lax
from jax.experimental import pallas as pl
from jax.experimental.pallas import tpu as pltpu
from jax.experimental.pallas import tpu_sc as plsc

_EXTRA = 16
_CHUNK = 1600
_B = 96
_BLKN = 2000
_BLKE = 4000


def _h1_body(h_ref, w_ref, o_ref):
    o_ref[...] = lax.dot_general(h_ref[...], w_ref[...], (((1,), (1,)), ((), ())),
                                 preferred_element_type=jnp.float32)


def _e1_body(a_ref, w_ref, b_ref, o_ref):
    o_ref[...] = lax.dot_general(a_ref[...], w_ref[...], (((1,), (1,)), ((), ())),
                                 preferred_element_type=jnp.float32) + b_ref[...]


def _fin_body(hagg_ref, deg_ref, h_ref, w2_ref, b2_ref,
              wih_ref, bih_ref, whh_ref, bhh_ref, g_ref, be_ref, mu_ref, var_ref,
              o_ref):
    D = h_ref.shape[1]
    h = h_ref[...]
    agg = lax.dot_general(hagg_ref[...], w2_ref[...], (((1,), (1,)), ((), ())),
                          preferred_element_type=jnp.float32) + deg_ref[...] * b2_ref[...]
    gi = lax.dot_general(agg, wih_ref[...], (((1,), (1,)), ((), ())),
                         preferred_element_type=jnp.float32) + bih_ref[...]
    gh = lax.dot_general(h, whh_ref[...], (((1,), (1,)), ((), ())),
                         preferred_element_type=jnp.float32) + bhh_ref[...]
    r = jax.nn.sigmoid(gi[:, :D] + gh[:, :D])
    z = jax.nn.sigmoid(gi[:, D:2 * D] + gh[:, D:2 * D])
    n = jnp.tanh(gi[:, 2 * D:] + r * gh[:, 2 * D:])
    h_new = (1.0 - z) * n + z * h
    bn = g_ref[...] * (h_new - mu_ref[...]) * lax.rsqrt(var_ref[...] + 1e-5) + be_ref[...]
    o_ref[...] = h + bn


def _make_sc_agg(N, E, D):
    info = plsc.get_sparse_core_info()
    NC, NS, L = info.num_cores, info.num_subcores, info.num_lanes
    NW = NC * NS
    W = D + L
    PASSES = 2
    NBUK = NW * PASSES
    RPT = ((N + NBUK - 1) // NBUK + 7) // 8 * 8
    NPAD = NBUK * RPT
    TRASHL = RPT
    CH = 1024
    B = 32
    NBC = CH // B
    mesh = plsc.VectorSubcoreMesh(core_axis_name="c", subcore_axis_name="s")

    @functools.partial(
        pl.kernel, mesh=mesh,
        out_type=jax.ShapeDtypeStruct((NPAD, W), jnp.float32),
        scratch_types=[
            pltpu.VMEM((CH,), jnp.int32),
            pltpu.VMEM((CH,), jnp.int32),
            pltpu.VMEM((CH,), jnp.int32),
            pltpu.VMEM((B,), jnp.int32),
            pltpu.VMEM((B,), jnp.int32),
            pltpu.VMEM((B,), jnp.int32),
            pltpu.VMEM((16,), jnp.int32),
            pltpu.VMEM((16,), jnp.int32),
            pltpu.VMEM((16,), jnp.int32),
            pltpu.VMEM((B, D), jnp.float32),
            pltpu.VMEM((B, D), jnp.float32),
            pltpu.VMEM((RPT + 1, W), jnp.float32),
            pltpu.SemaphoreType.DMA,
            pltpu.SemaphoreType.DMA,
        ],
    )
    def sc_agg(h1_hbm, e1_hbm, src_hbm, dst_hbm, eid_hbm, lo_hbm, hi_hbm, rb_hbm,
               out_hbm, srcc, dstc, eidc, gsrc, geid, sloc, lov, hiv, rbv,
               bufh, bufe, aggl, semh, seme):
        cid = lax.axis_index("c")
        sid = lax.axis_index("s")
        wid = sid * NC + cid
        iota = lax.iota(jnp.int32, L)
        zero16 = jnp.zeros((L,), jnp.float32)
        onecol = jnp.where(iota == 0, jnp.float32(1.0), jnp.float32(0.0))

        for paz in range(PASSES):
            buk = paz * NW + wid

            def zrow(rr, _):
                for k in range(W // L):
                    aggl[rr, pl.ds(k * L, L)] = zero16
                return 0
            lax.fori_loop(0, RPT + 1, zrow, 0)

            pltpu.sync_copy(lo_hbm.at[pl.ds(buk * L, L)], lov)
            pltpu.sync_copy(hi_hbm.at[pl.ds(buk * L, L)], hiv)
            pltpu.sync_copy(rb_hbm.at[pl.ds(buk * L, L)], rbv)
            lovec = lov[pl.ds(0, L)]
            hivec = hiv[pl.ds(0, L)]
            rbvec = rbv[pl.ds(0, L)]
            lo8vec = lovec - lovec % 8
            lo8 = pl.multiple_of(lo8vec[0], 8)
            hi = hivec[0]
            nch = (hi - lo8 + CH - 1) // CH

            def chunk_body(ch, _):
                ebase = pl.multiple_of(lo8 + ch * CH, 8)
                pltpu.sync_copy(src_hbm.at[pl.ds(ebase, CH)], srcc)
                pltpu.sync_copy(dst_hbm.at[pl.ds(ebase, CH)], dstc)
                pltpu.sync_copy(eid_hbm.at[pl.ds(ebase, CH)], eidc)
                gi0 = lo8vec + ch * CH

                def batch_body(bj, _):
                    bb = bj * B
                    for t in range(B // L):
                        giv = gi0 + (bb + t * L) + iota
                        valid = (giv >= lovec) & (giv < hivec)
                        dv = dstc[pl.ds(bb + t * L, L)]
                        sv = srcc[pl.ds(bb + t * L, L)]
                        ev = eidc[pl.ds(bb + t * L, L)]
                        gsrc[pl.ds(t * L, L)] = jnp.where(valid, sv, 0)
                        geid[pl.ds(t * L, L)] = jnp.where(valid, ev, 0)
                        sloc[pl.ds(t * L, L)] = jnp.where(valid, dv - rbvec, TRASHL)
                    ca = pltpu.async_copy(h1_hbm.at[gsrc], bufh, semh)
                    cb = pltpu.async_copy(e1_hbm.at[geid], bufe, seme)
                    ca.wait()
                    cb.wait()

                    def acc16(q, _):
                        locv = sloc[pl.ds(q * L, L)]
                        for lane in range(L):
                            loc = locv[lane]
                            i = q * L + lane
                            dc = aggl[loc, pl.ds(D, L)]
                            aggl[loc, pl.ds(D, L)] = dc + onecol
                            for k in range(D // L):
                                v = bufh[i, pl.ds(k * L, L)] + bufe[i, pl.ds(k * L, L)]
                                a = aggl[loc, pl.ds(k * L, L)]
                                aggl[loc, pl.ds(k * L, L)] = a + jnp.maximum(v, 0.0)
                        return 0
                    lax.fori_loop(0, B // L, acc16, 0)
                    return 0

                lax.fori_loop(0, NBC, batch_body, 0)
                return 0

            lax.fori_loop(0, nch, chunk_body, 0)
            pltpu.sync_copy(aggl.at[pl.ds(0, RPT)], out_hbm.at[pl.ds(buk * RPT, RPT)])

    return sc_agg, NPAD, RPT


def kernel(h, edge_index, edge_attr, W1, b1, W2, b2, bn_gamma, bn_beta,
           bn_mean, bn_var, W_ih, W_hh, b_ih, b_hh):
    N, D = h.shape
    E, DE = edge_attr.shape
    ei = edge_index.astype(jnp.int32)
    src, dst = ei[0], ei[1]
    W1a = W1[:, :D]
    W1b = W1[:, D:]

    gN = N // _BLKN
    H1 = pl.pallas_call(
        _h1_body,
        grid=(gN,),
        in_specs=[pl.BlockSpec((_BLKN, D), lambda i: (i, 0)),
                  pl.BlockSpec((D, D), lambda i: (0, 0))],
        out_specs=pl.BlockSpec((_BLKN, D), lambda i: (i, 0)),
        out_shape=jax.ShapeDtypeStruct((N, D), jnp.float32),
    )(h, W1a)

    gE = E // _BLKE
    E1 = pl.pallas_call(
        _e1_body,
        grid=(gE,),
        in_specs=[pl.BlockSpec((_BLKE, DE), lambda i: (i, 0)),
                  pl.BlockSpec((D, DE), lambda i: (0, 0)),
                  pl.BlockSpec((1, D), lambda i: (0, 0))],
        out_specs=pl.BlockSpec((_BLKE, D), lambda i: (i, 0)),
        out_shape=jax.ShapeDtypeStruct((E, D), jnp.float32),
    )(edge_attr, W1b, b1.reshape(1, D))

    sc_agg, NPAD, RPT = _make_sc_agg(N, E, D)
    NBUK = NPAD // RPT
    CH = 1024
    bucket = dst // RPT
    perm = jnp.argsort(bucket, stable=True).astype(jnp.int32)
    pad = jnp.zeros((CH,), jnp.int32)
    srcp = jnp.concatenate([src[perm], pad])
    dstp = jnp.concatenate([dst[perm], pad])
    eidp = jnp.concatenate([perm, pad])
    off = jnp.searchsorted(bucket[perm], jnp.arange(NBUK + 1), side="left").astype(jnp.int32)
    lo_tab = jnp.repeat(off[:NBUK], 16)
    hi_tab = jnp.repeat(off[1:NBUK + 1], 16)
    rb_tab = jnp.repeat(jnp.arange(NBUK, dtype=jnp.int32) * RPT, 16)
    haggw = sc_agg(H1, E1, srcp, dstp, eidp, lo_tab, hi_tab, rb_tab)
    hagg0 = haggw[:N, :D]
    deg0 = haggw[:N, D:D + 1]

    D3 = 3 * D
    out = pl.pallas_call(
        _fin_body,
        grid=(gN,),
        in_specs=[pl.BlockSpec((_BLKN, D), lambda i: (i, 0)),
                  pl.BlockSpec((_BLKN, 1), lambda i: (i, 0)),
                  pl.BlockSpec((_BLKN, D), lambda i: (i, 0)),
                  pl.BlockSpec((D, D), lambda i: (0, 0)),
                  pl.BlockSpec((1, D), lambda i: (0, 0)),
                  pl.BlockSpec((D3, D), lambda i: (0, 0)),
                  pl.BlockSpec((1, D3), lambda i: (0, 0)),
                  pl.BlockSpec((D3, D), lambda i: (0, 0)),
                  pl.BlockSpec((1, D3), lambda i: (0, 0)),
                  pl.BlockSpec((1, D), lambda i: (0, 0)),
                  pl.BlockSpec((1, D), lambda i: (0, 0)),
                  pl.BlockSpec((1, D), lambda i: (0, 0)),
                  pl.BlockSpec((1, D), lambda i: (0, 0))],
        out_specs=pl.BlockSpec((_BLKN, D), lambda i: (i, 0)),
        out_shape=jax.ShapeDtypeStruct((N, D), jnp.float32),
    )(hagg0, deg0, h, W2, b2.reshape(1, D), W_ih, b_ih.reshape(1, D3),
      W_hh, b_hh.reshape(1, D3), bn_gamma.reshape(1, D), bn_beta.reshape(1, D),
      bn_mean.reshape(1, D), bn_var.reshape(1, D))
    return out


# speedup vs baseline: 3.9446x; 1.3438x over previous
import functools

import jax
import jax.numpy as jnp
from jax import lax
from jax.experimental import pallas as pl
from jax.experimental.pallas import tpu as pltpu
from jax.experimental.pallas import tpu_sc as plsc

_EXTRA = 16
_CHUNK = 1600
_B = 96
_BLKN = 2000
_BLKE = 4000


def _h1_body(h_ref, w_ref, o_ref):
    o_ref[...] = lax.dot_general(h_ref[...], w_ref[...], (((1,), (1,)), ((), ())),
                                 preferred_element_type=jnp.float32)


def _e1_body(a_ref, w_ref, b_ref, o_ref):
    o_ref[...] = lax.dot_general(a_ref[...], w_ref[...], (((1,), (1,)), ((), ())),
                                 preferred_element_type=jnp.float32) + b_ref[...]


def _fin_body(hagg_ref, deg_ref, h_ref, w2_ref, b2_ref,
              wih_ref, bih_ref, whh_ref, bhh_ref, g_ref, be_ref, mu_ref, var_ref,
              o_ref):
    D = h_ref.shape[1]
    h = h_ref[...]
    agg = lax.dot_general(hagg_ref[...], w2_ref[...], (((1,), (1,)), ((), ())),
                          preferred_element_type=jnp.float32) + deg_ref[...] * b2_ref[...]
    gi = lax.dot_general(agg, wih_ref[...], (((1,), (1,)), ((), ())),
                         preferred_element_type=jnp.float32) + bih_ref[...]
    gh = lax.dot_general(h, whh_ref[...], (((1,), (1,)), ((), ())),
                         preferred_element_type=jnp.float32) + bhh_ref[...]
    r = jax.nn.sigmoid(gi[:, :D] + gh[:, :D])
    z = jax.nn.sigmoid(gi[:, D:2 * D] + gh[:, D:2 * D])
    n = jnp.tanh(gi[:, 2 * D:] + r * gh[:, 2 * D:])
    h_new = (1.0 - z) * n + z * h
    bn = g_ref[...] * (h_new - mu_ref[...]) * lax.rsqrt(var_ref[...] + 1e-5) + be_ref[...]
    o_ref[...] = h + bn


def _make_sc_agg(N, E, D):
    info = plsc.get_sparse_core_info()
    NC, NS, L = info.num_cores, info.num_subcores, info.num_lanes
    NW = NC * NS
    W = D + L
    PASSES = 2
    NBUK = NW * PASSES
    RPT = ((N + NBUK - 1) // NBUK + 7) // 8 * 8
    NPAD = NBUK * RPT
    TRASHL = RPT
    CH = 1024
    B = 32
    NBC = CH // B
    mesh = plsc.VectorSubcoreMesh(core_axis_name="c", subcore_axis_name="s")

    @functools.partial(
        pl.kernel, mesh=mesh,
        out_type=jax.ShapeDtypeStruct((NPAD, W), jnp.float32),
        scratch_types=[
            pltpu.VMEM((CH,), jnp.int32),
            pltpu.VMEM((CH,), jnp.int32),
            pltpu.VMEM((CH,), jnp.int32),
            pltpu.VMEM((2, B), jnp.int32),
            pltpu.VMEM((2, B), jnp.int32),
            pltpu.VMEM((2, B), jnp.int32),
            pltpu.VMEM((16,), jnp.int32),
            pltpu.VMEM((16,), jnp.int32),
            pltpu.VMEM((16,), jnp.int32),
            pltpu.VMEM((2, B, D), jnp.float32),
            pltpu.VMEM((2, B, D), jnp.float32),
            pltpu.VMEM((RPT + 1, W), jnp.float32),
            pltpu.SemaphoreType.DMA,
            pltpu.SemaphoreType.DMA,
            pltpu.SemaphoreType.DMA,
            pltpu.SemaphoreType.DMA,
        ],
    )
    def sc_agg(h1_hbm, e1_hbm, src_hbm, dst_hbm, eid_hbm, lo_hbm, hi_hbm, rb_hbm,
               out_hbm, srcc, dstc, eidc, gsrc, geid, sloc, lov, hiv, rbv,
               bufh, bufe, aggl, semh0, seme0, semh1, seme1):
        cid = lax.axis_index("c")
        sid = lax.axis_index("s")
        wid = sid * NC + cid
        iota = lax.iota(jnp.int32, L)
        zero16 = jnp.zeros((L,), jnp.float32)
        onecol = jnp.where(iota == 0, jnp.float32(1.0), jnp.float32(0.0))

        def pass_body(paz, _):
            buk = paz * NW + wid

            def zrow(rr, _):
                for k in range(W // L):
                    aggl[rr, pl.ds(k * L, L)] = zero16
                return 0
            lax.fori_loop(0, RPT + 1, zrow, 0)

            pltpu.sync_copy(lo_hbm.at[pl.ds(buk * L, L)], lov)
            pltpu.sync_copy(hi_hbm.at[pl.ds(buk * L, L)], hiv)
            pltpu.sync_copy(rb_hbm.at[pl.ds(buk * L, L)], rbv)
            lovec = lov[pl.ds(0, L)]
            hivec = hiv[pl.ds(0, L)]
            rbvec = rbv[pl.ds(0, L)]
            lo8vec = lovec - lovec % 8
            lo8 = pl.multiple_of(lo8vec[0], 8)
            hi = hivec[0]
            nch = (hi - lo8 + CH - 1) // CH

            nbt = (hi - lo8 + B - 1) // B
            sems = ((semh0, seme0), (semh1, seme1))

            def stage_fire(bg, slot):
                base = pl.multiple_of(lo8 + bg * B, 8)
                pltpu.sync_copy(src_hbm.at[pl.ds(base, B)], srcc.at[pl.ds(0, B)])
                pltpu.sync_copy(dst_hbm.at[pl.ds(base, B)], dstc.at[pl.ds(0, B)])
                pltpu.sync_copy(eid_hbm.at[pl.ds(base, B)], eidc.at[pl.ds(0, B)])
                for t in range(B // L):
                    giv = lo8vec + bg * B + t * L + iota
                    valid = (giv >= lovec) & (giv < hivec)
                    dv = dstc[pl.ds(t * L, L)]
                    sv = srcc[pl.ds(t * L, L)]
                    ev = eidc[pl.ds(t * L, L)]
                    gsrc[slot, pl.ds(t * L, L)] = jnp.where(valid, sv, 0)
                    geid[slot, pl.ds(t * L, L)] = jnp.where(valid, ev, 0)
                    sloc[slot, pl.ds(t * L, L)] = jnp.where(valid, dv - rbvec, TRASHL)
                pltpu.async_copy(h1_hbm.at[gsrc.at[slot]], bufh.at[slot], sems[slot][0])
                pltpu.async_copy(e1_hbm.at[geid.at[slot]], bufe.at[slot], sems[slot][1])

            def drain(slot):
                pltpu.make_async_copy(h1_hbm.at[gsrc.at[slot]], bufh.at[slot],
                                      sems[slot][0]).wait()
                pltpu.make_async_copy(e1_hbm.at[geid.at[slot]], bufe.at[slot],
                                      sems[slot][1]).wait()

            def accum(slot):
                def acc16(q, _):
                    locv = sloc[slot, pl.ds(q * L, L)]
                    for lane in range(L):
                        loc = locv[lane]
                        i = q * L + lane
                        dc = aggl[loc, pl.ds(D, L)]
                        aggl[loc, pl.ds(D, L)] = dc + onecol
                        for k in range(D // L):
                            v = (bufh[slot, i, pl.ds(k * L, L)]
                                 + bufe[slot, i, pl.ds(k * L, L)])
                            a = aggl[loc, pl.ds(k * L, L)]
                            aggl[loc, pl.ds(k * L, L)] = a + jnp.maximum(v, 0.0)
                    return 0
                lax.fori_loop(0, B // L, acc16, 0)

            stage_fire(0, 0)
            drain(0)

            def pair_body(g2, _):
                b0 = 2 * g2
                stage_fire(b0 + 1, 1)
                accum(0)
                drain(1)
                stage_fire(b0 + 2, 0)
                accum(1)
                drain(0)
                return 0

            lax.fori_loop(0, (nbt + 1) // 2, pair_body, 0)
            pltpu.sync_copy(aggl.at[pl.ds(0, RPT)],
                            out_hbm.at[pl.ds(pl.multiple_of(buk * RPT, 8), RPT)])
            return 0

        lax.fori_loop(0, PASSES, pass_body, 0)

    return sc_agg, NPAD, RPT


def kernel(h, edge_index, edge_attr, W1, b1, W2, b2, bn_gamma, bn_beta,
           bn_mean, bn_var, W_ih, W_hh, b_ih, b_hh):
    N, D = h.shape
    E, DE = edge_attr.shape
    ei = edge_index.astype(jnp.int32)
    src, dst = ei[0], ei[1]
    W1a = W1[:, :D]
    W1b = W1[:, D:]

    gN = N // _BLKN
    H1 = pl.pallas_call(
        _h1_body,
        grid=(gN,),
        in_specs=[pl.BlockSpec((_BLKN, D), lambda i: (i, 0)),
                  pl.BlockSpec((D, D), lambda i: (0, 0))],
        out_specs=pl.BlockSpec((_BLKN, D), lambda i: (i, 0)),
        out_shape=jax.ShapeDtypeStruct((N, D), jnp.float32),
    )(h, W1a)

    gE = E // _BLKE
    E1 = pl.pallas_call(
        _e1_body,
        grid=(gE,),
        in_specs=[pl.BlockSpec((_BLKE, DE), lambda i: (i, 0)),
                  pl.BlockSpec((D, DE), lambda i: (0, 0)),
                  pl.BlockSpec((1, D), lambda i: (0, 0))],
        out_specs=pl.BlockSpec((_BLKE, D), lambda i: (i, 0)),
        out_shape=jax.ShapeDtypeStruct((E, D), jnp.float32),
    )(edge_attr, W1b, b1.reshape(1, D))

    sc_agg, NPAD, RPT = _make_sc_agg(N, E, D)
    NBUK = NPAD // RPT
    CH = 1024
    bucket = dst // RPT
    perm = jnp.argsort(bucket, stable=True).astype(jnp.int32)
    pad = jnp.zeros((CH,), jnp.int32)
    srcp = jnp.concatenate([src[perm], pad])
    dstp = jnp.concatenate([dst[perm], pad])
    eidp = jnp.concatenate([perm, pad])
    off = jnp.searchsorted(bucket[perm], jnp.arange(NBUK + 1), side="left").astype(jnp.int32)
    lo_tab = jnp.repeat(off[:NBUK], 16)
    hi_tab = jnp.repeat(off[1:NBUK + 1], 16)
    rb_tab = jnp.repeat(jnp.arange(NBUK, dtype=jnp.int32) * RPT, 16)
    haggw = sc_agg(H1, E1, srcp, dstp, eidp, lo_tab, hi_tab, rb_tab)
    hagg0 = haggw[:N, :D]
    deg0 = haggw[:N, D:D + 1]

    D3 = 3 * D
    out = pl.pallas_call(
        _fin_body,
        grid=(gN,),
        in_specs=[pl.BlockSpec((_BLKN, D), lambda i: (i, 0)),
                  pl.BlockSpec((_BLKN, 1), lambda i: (i, 0)),
                  pl.BlockSpec((_BLKN, D), lambda i: (i, 0)),
                  pl.BlockSpec((D, D), lambda i: (0, 0)),
                  pl.BlockSpec((1, D), lambda i: (0, 0)),
                  pl.BlockSpec((D3, D), lambda i: (0, 0)),
                  pl.BlockSpec((1, D3), lambda i: (0, 0)),
                  pl.BlockSpec((D3, D), lambda i: (0, 0)),
                  pl.BlockSpec((1, D3), lambda i: (0, 0)),
                  pl.BlockSpec((1, D), lambda i: (0, 0)),
                  pl.BlockSpec((1, D), lambda i: (0, 0)),
                  pl.BlockSpec((1, D), lambda i: (0, 0)),
                  pl.BlockSpec((1, D), lambda i: (0, 0))],
        out_specs=pl.BlockSpec((_BLKN, D), lambda i: (i, 0)),
        out_shape=jax.ShapeDtypeStruct((N, D), jnp.float32),
    )(hagg0, deg0, h, W2, b2.reshape(1, D), W_ih, b_ih.reshape(1, D3),
      W_hh, b_hh.reshape(1, D3), bn_gamma.reshape(1, D), bn_beta.reshape(1, D),
      bn_mean.reshape(1, D), bn_var.reshape(1, D))
    return out


# speedup vs baseline: 4.0766x; 1.0335x over previous
import functools

import jax
import jax.numpy as jnp
from jax import lax
from jax.experimental import pallas as pl
from jax.experimental.pallas import tpu as pltpu
from jax.experimental.pallas import tpu_sc as plsc

_EXTRA = 16
_CHUNK = 1600
_B = 96
_BLKN = 2000
_BLKE = 4000


def _h1_body(h_ref, w_ref, o_ref):
    o_ref[...] = lax.dot_general(h_ref[...], w_ref[...], (((1,), (1,)), ((), ())),
                                 preferred_element_type=jnp.float32)


def _e1_body(a_ref, w_ref, b_ref, o_ref):
    o_ref[...] = lax.dot_general(a_ref[...], w_ref[...], (((1,), (1,)), ((), ())),
                                 preferred_element_type=jnp.float32) + b_ref[...]


def _fin_body(hagg_ref, deg_ref, h_ref, w2_ref, b2_ref,
              wih_ref, bih_ref, whh_ref, bhh_ref, g_ref, be_ref, mu_ref, var_ref,
              o_ref):
    D = h_ref.shape[1]
    h = h_ref[...]
    agg = lax.dot_general(hagg_ref[...], w2_ref[...], (((1,), (1,)), ((), ())),
                          preferred_element_type=jnp.float32) + deg_ref[...] * b2_ref[...]
    gi = lax.dot_general(agg, wih_ref[...], (((1,), (1,)), ((), ())),
                         preferred_element_type=jnp.float32) + bih_ref[...]
    gh = lax.dot_general(h, whh_ref[...], (((1,), (1,)), ((), ())),
                         preferred_element_type=jnp.float32) + bhh_ref[...]
    r = jax.nn.sigmoid(gi[:, :D] + gh[:, :D])
    z = jax.nn.sigmoid(gi[:, D:2 * D] + gh[:, D:2 * D])
    n = jnp.tanh(gi[:, 2 * D:] + r * gh[:, 2 * D:])
    h_new = (1.0 - z) * n + z * h
    bn = g_ref[...] * (h_new - mu_ref[...]) * lax.rsqrt(var_ref[...] + 1e-5) + be_ref[...]
    o_ref[...] = h + bn


def _make_sc_agg(N, E, D):
    info = plsc.get_sparse_core_info()
    NC, NS, L = info.num_cores, info.num_subcores, info.num_lanes
    NW = NC * NS
    W = D + L
    PASSES = 2
    NBUK = NW * PASSES
    RPT = ((N + NBUK - 1) // NBUK + 7) // 8 * 8
    NPAD = NBUK * RPT
    TRASHL = RPT
    CH = 1024
    B = 48
    NBC = CH // B
    mesh = plsc.VectorSubcoreMesh(core_axis_name="c", subcore_axis_name="s")

    @functools.partial(
        pl.kernel, mesh=mesh,
        out_type=jax.ShapeDtypeStruct((NPAD, D), jnp.float32),
        scratch_types=[
            pltpu.VMEM((CH,), jnp.int32),
            pltpu.VMEM((CH,), jnp.int32),
            pltpu.VMEM((CH,), jnp.int32),
            pltpu.VMEM((2, B), jnp.int32),
            pltpu.VMEM((2, B), jnp.int32),
            pltpu.VMEM((2, B), jnp.int32),
            pltpu.VMEM((16,), jnp.int32),
            pltpu.VMEM((16,), jnp.int32),
            pltpu.VMEM((16,), jnp.int32),
            pltpu.VMEM((2, B, D), jnp.float32),
            pltpu.VMEM((2, B, D), jnp.float32),
            pltpu.VMEM((RPT + 1, D), jnp.float32),
            pltpu.SemaphoreType.DMA,
            pltpu.SemaphoreType.DMA,
            pltpu.SemaphoreType.DMA,
            pltpu.SemaphoreType.DMA,
        ],
    )
    def sc_agg(h1_hbm, e1_hbm, src_hbm, dst_hbm, eid_hbm, lo_hbm, hi_hbm, rb_hbm,
               out_hbm, srcc, dstc, eidc, gsrc, geid, sloc, lov, hiv, rbv,
               bufh, bufe, aggl, semh0, seme0, semh1, seme1):
        cid = lax.axis_index("c")
        sid = lax.axis_index("s")
        wid = sid * NC + cid
        iota = lax.iota(jnp.int32, L)
        zero16 = jnp.zeros((L,), jnp.float32)
        onecol = jnp.where(iota == 0, jnp.float32(1.0), jnp.float32(0.0))

        def pass_body(paz, _):
            buk = paz * NW + wid

            def zrow(rr, _):
                for k in range(D // L):
                    aggl[rr, pl.ds(k * L, L)] = zero16
                return 0
            lax.fori_loop(0, RPT + 1, zrow, 0)

            pltpu.sync_copy(lo_hbm.at[pl.ds(buk * L, L)], lov)
            pltpu.sync_copy(hi_hbm.at[pl.ds(buk * L, L)], hiv)
            pltpu.sync_copy(rb_hbm.at[pl.ds(buk * L, L)], rbv)
            lovec = lov[pl.ds(0, L)]
            hivec = hiv[pl.ds(0, L)]
            rbvec = rbv[pl.ds(0, L)]
            lo8vec = lovec - lovec % 8
            lo8 = pl.multiple_of(lo8vec[0], 8)
            hi = hivec[0]
            nch = (hi - lo8 + CH - 1) // CH

            nbt = (hi - lo8 + B - 1) // B
            sems = ((semh0, seme0), (semh1, seme1))

            def stage_fire(bg, slot):
                base = pl.multiple_of(lo8 + bg * B, 8)
                pltpu.sync_copy(src_hbm.at[pl.ds(base, B)], srcc.at[pl.ds(0, B)])
                pltpu.sync_copy(dst_hbm.at[pl.ds(base, B)], dstc.at[pl.ds(0, B)])
                pltpu.sync_copy(eid_hbm.at[pl.ds(base, B)], eidc.at[pl.ds(0, B)])
                for t in range(B // L):
                    giv = lo8vec + bg * B + t * L + iota
                    valid = (giv >= lovec) & (giv < hivec)
                    dv = dstc[pl.ds(t * L, L)]
                    sv = srcc[pl.ds(t * L, L)]
                    ev = eidc[pl.ds(t * L, L)]
                    gsrc[slot, pl.ds(t * L, L)] = jnp.where(valid, sv, 0)
                    geid[slot, pl.ds(t * L, L)] = jnp.where(valid, ev, 0)
                    sloc[slot, pl.ds(t * L, L)] = jnp.where(valid, dv - rbvec, TRASHL)
                pltpu.async_copy(h1_hbm.at[gsrc.at[slot]], bufh.at[slot], sems[slot][0])
                pltpu.async_copy(e1_hbm.at[geid.at[slot]], bufe.at[slot], sems[slot][1])

            def drain(slot):
                pltpu.make_async_copy(h1_hbm.at[gsrc.at[slot]], bufh.at[slot],
                                      sems[slot][0]).wait()
                pltpu.make_async_copy(e1_hbm.at[geid.at[slot]], bufe.at[slot],
                                      sems[slot][1]).wait()

            def accum(slot):
                def acc16(q, _):
                    locv = sloc[slot, pl.ds(q * L, L)]
                    for lane in range(L):
                        loc = locv[lane]
                        i = q * L + lane
                        for k in range(D // L):
                            v = (bufh[slot, i, pl.ds(k * L, L)]
                                 + bufe[slot, i, pl.ds(k * L, L)])
                            a = aggl[loc, pl.ds(k * L, L)]
                            aggl[loc, pl.ds(k * L, L)] = a + jnp.maximum(v, 0.0)
                    return 0
                lax.fori_loop(0, B // L, acc16, 0)

            stage_fire(0, 0)
            drain(0)

            def pair_body(g2, _):
                b0 = 2 * g2
                stage_fire(b0 + 1, 1)
                accum(0)
                drain(1)
                stage_fire(b0 + 2, 0)
                accum(1)
                drain(0)
                return 0

            lax.fori_loop(0, (nbt + 1) // 2, pair_body, 0)
            pltpu.sync_copy(aggl.at[pl.ds(0, RPT)],
                            out_hbm.at[pl.ds(pl.multiple_of(buk * RPT, 8), RPT)])
            return 0

        lax.fori_loop(0, PASSES, pass_body, 0)

    return sc_agg, NPAD, RPT


def kernel(h, edge_index, edge_attr, W1, b1, W2, b2, bn_gamma, bn_beta,
           bn_mean, bn_var, W_ih, W_hh, b_ih, b_hh):
    N, D = h.shape
    E, DE = edge_attr.shape
    ei = edge_index.astype(jnp.int32)
    src, dst = ei[0], ei[1]
    W1a = W1[:, :D]
    W1b = W1[:, D:]

    gN = N // _BLKN
    H1 = pl.pallas_call(
        _h1_body,
        grid=(gN,),
        in_specs=[pl.BlockSpec((_BLKN, D), lambda i: (i, 0)),
                  pl.BlockSpec((D, D), lambda i: (0, 0))],
        out_specs=pl.BlockSpec((_BLKN, D), lambda i: (i, 0)),
        out_shape=jax.ShapeDtypeStruct((N, D), jnp.float32),
    )(h, W1a)

    gE = E // _BLKE
    E1 = pl.pallas_call(
        _e1_body,
        grid=(gE,),
        in_specs=[pl.BlockSpec((_BLKE, DE), lambda i: (i, 0)),
                  pl.BlockSpec((D, DE), lambda i: (0, 0)),
                  pl.BlockSpec((1, D), lambda i: (0, 0))],
        out_specs=pl.BlockSpec((_BLKE, D), lambda i: (i, 0)),
        out_shape=jax.ShapeDtypeStruct((E, D), jnp.float32),
    )(edge_attr, W1b, b1.reshape(1, D))

    sc_agg, NPAD, RPT = _make_sc_agg(N, E, D)
    NBUK = NPAD // RPT
    CH = 1024
    perm = jnp.argsort(dst).astype(jnp.int32)
    pad = jnp.zeros((CH,), jnp.int32)
    srcp = jnp.concatenate([src[perm], pad])
    dstp = jnp.concatenate([dst[perm], pad])
    eidp = jnp.concatenate([perm, pad])
    off = jnp.searchsorted(dst[perm], jnp.arange(NBUK + 1) * RPT, side="left").astype(jnp.int32)
    lo_tab = jnp.repeat(off[:NBUK], 16)
    hi_tab = jnp.repeat(off[1:NBUK + 1], 16)
    rb_tab = jnp.repeat(jnp.arange(NBUK, dtype=jnp.int32) * RPT, 16)
    noff = jnp.searchsorted(dst[perm], jnp.arange(N + 1), side="left").astype(jnp.int32)
    haggw = sc_agg(H1, E1, srcp, dstp, eidp, lo_tab, hi_tab, rb_tab)
    hagg0 = haggw[:N, :D]
    deg0 = (noff[1:] - noff[:-1]).astype(jnp.float32).reshape(N, 1)

    D3 = 3 * D
    out = pl.pallas_call(
        _fin_body,
        grid=(gN,),
        in_specs=[pl.BlockSpec((_BLKN, D), lambda i: (i, 0)),
                  pl.BlockSpec((_BLKN, 1), lambda i: (i, 0)),
                  pl.BlockSpec((_BLKN, D), lambda i: (i, 0)),
                  pl.BlockSpec((D, D), lambda i: (0, 0)),
                  pl.BlockSpec((1, D), lambda i: (0, 0)),
                  pl.BlockSpec((D3, D), lambda i: (0, 0)),
                  pl.BlockSpec((1, D3), lambda i: (0, 0)),
                  pl.BlockSpec((D3, D), lambda i: (0, 0)),
                  pl.BlockSpec((1, D3), lambda i: (0, 0)),
                  pl.BlockSpec((1, D), lambda i: (0, 0)),
                  pl.BlockSpec((1, D), lambda i: (0, 0)),
                  pl.BlockSpec((1, D), lambda i: (0, 0)),
                  pl.BlockSpec((1, D), lambda i: (0, 0))],
        out_specs=pl.BlockSpec((_BLKN, D), lambda i: (i, 0)),
        out_shape=jax.ShapeDtypeStruct((N, D), jnp.float32),
    )(hagg0, deg0, h, W2, b2.reshape(1, D), W_ih, b_ih.reshape(1, D3),
      W_hh, b_hh.reshape(1, D3), bn_gamma.reshape(1, D), bn_beta.reshape(1, D),
      bn_mean.reshape(1, D), bn_var.reshape(1, D))
    return out
